# baseline (speedup 1.0000x reference)
import jax
import jax.numpy as jnp
from jax import lax
from jax.experimental import pallas as pl
from jax.experimental.pallas import tpu as pltpu
from jax.experimental.pallas import tpu_sc as plsc

HIDDEN = 128
MAX_NB = 16
N_ATOMS = 10000
N_BONDS = 160000
N_MOLS = 100
ATOMS_PER_MOL = N_ATOMS // N_MOLS
N_ATOMS_PAD = 10240

NC, NS = 2, 16
NW = NC * NS


def _make_gather_sum(n_rows_out, chunk, nbuf, name):
  assert n_rows_out % chunk == 0 and chunk <= 128 and chunk % 8 == 0
  total_chunks = n_rows_out // chunk
  n_iter = -(-total_chunks // NW)
  n_outer = -(-n_iter // nbuf)
  mesh = plsc.VectorSubcoreMesh(
      core_axis_name="c", subcore_axis_name="s", num_cores=NC, num_subcores=NS
  )

  def body(table_hbm, idxt_hbm, out_hbm, idx_v, acc_v, gsem, osem):
    wid = lax.axis_index("s") * NC + lax.axis_index("c")

    def outer_body(j, carry):
      for b in range(nbuf):
        c = (nbuf * j + b) * NW + wid

        @pl.when(jnp.logical_and(j > 0, c - nbuf * NW < total_chunks))
        def _(b=b):
          pltpu.make_async_copy(
              out_hbm.at[pl.ds(0, chunk)], acc_v.at[b], osem[b]
          ).wait()

        @pl.when(c < total_chunks)
        def _(b=b, c=c):
          base = c * chunk
          pltpu.sync_copy(idxt_hbm.at[:, pl.ds(base, chunk)], idx_v.at[b])
          pltpu.async_copy(
              table_hbm.at[idx_v.at[b, 0]], acc_v.at[b], gsem[b]
          ).wait()
          for k in range(1, MAX_NB):
            pltpu.async_copy(
                table_hbm.at[idx_v.at[b, k]], acc_v.at[b], gsem[b], add=True
            )

      for b in range(nbuf):
        c = (nbuf * j + b) * NW + wid

        @pl.when(c < total_chunks)
        def _(b=b, c=c):
          for _k in range(1, MAX_NB):
            pltpu.make_async_copy(
                table_hbm.at[idx_v.at[b, 0]], acc_v.at[b], gsem[b]
            ).wait()
          pltpu.async_copy(acc_v.at[b], out_hbm.at[pl.ds(c * chunk, chunk)],
                           osem[b])

      return carry

    lax.fori_loop(0, n_outer, outer_body, 0)

    for b in range(nbuf):
      c_last = (nbuf * (n_outer - 1) + b) * NW + wid

      @pl.when(c_last < total_chunks)
      def _(b=b):
        pltpu.make_async_copy(
            out_hbm.at[pl.ds(0, chunk)], acc_v.at[b], osem[b]
        ).wait()

  return pl.kernel(
      body,
      out_type=jax.ShapeDtypeStruct((n_rows_out, HIDDEN), jnp.float32),
      mesh=mesh,
      scratch_types=[
          pltpu.VMEM((nbuf, MAX_NB, chunk), jnp.int32),
          pltpu.VMEM((nbuf, chunk, HIDDEN), jnp.float32),
          [pltpu.SemaphoreType.DMA] * nbuf,
          [pltpu.SemaphoreType.DMA] * nbuf,
      ],
      name=name,
  )


_gather_cache = {}


def _gather_sum(n_rows_out, chunk, nbuf, name):
  key = (n_rows_out, chunk, nbuf, name)
  if key not in _gather_cache:
    _gather_cache[key] = _make_gather_sum(n_rows_out, chunk, nbuf, name)
  return _gather_cache[key]


_MM_ROWS = 4000


def _binput_body(fb_ref, wi_ref, bi_ref, msg_ref, junk_ref):
  bi = jnp.dot(fb_ref[...], wi_ref[...], preferred_element_type=jnp.float32)
  bi_ref[...] = bi
  msg_ref[...] = jnp.maximum(bi, 0.0)
  del junk_ref


_binput_mm = pl.pallas_call(
    _binput_body,
    grid=(N_BONDS // _MM_ROWS,),
    in_specs=[
        pl.BlockSpec((_MM_ROWS, HIDDEN), lambda i: (i, 0)),
        pl.BlockSpec((HIDDEN, HIDDEN), lambda i: (0, 0)),
    ],
    out_specs=[
        pl.BlockSpec((_MM_ROWS, HIDDEN), lambda i: (i, 0)),
        pl.BlockSpec((_MM_ROWS, HIDDEN), lambda i: (i, 0)),
        pl.BlockSpec(memory_space=pl.ANY),
    ],
    out_shape=[
        jax.ShapeDtypeStruct((N_BONDS, HIDDEN), jnp.float32),
        jax.ShapeDtypeStruct((N_BONDS, HIDDEN), jnp.float32),
        jax.ShapeDtypeStruct((N_BONDS, HIDDEN), jnp.float32),
    ],
)


N_HALF = N_BONDS // 2


def _round_half_body(dead_ref, bi_ref, nei_ref, wh_ref, out_ref):
  del dead_ref
  acc = jnp.dot(nei_ref[...], wh_ref[...], preferred_element_type=jnp.float32)
  out_ref[...] = jnp.maximum(bi_ref[...] + acc, 0.0)


def _make_round_half(half):
  nb = N_HALF // _MM_ROWS

  return pl.pallas_call(
      _round_half_body,
      grid=(nb,),
      in_specs=[
          pl.BlockSpec(memory_space=pl.ANY),
          pl.BlockSpec((_MM_ROWS, HIDDEN), lambda i: (i + half * nb, 0)),
          pl.BlockSpec((_MM_ROWS, HIDDEN), lambda i: (i, 0)),
          pl.BlockSpec((HIDDEN, HIDDEN), lambda i: (0, 0)),
      ],
      out_specs=pl.BlockSpec((_MM_ROWS, HIDDEN), lambda i: (i + half * nb, 0)),
      out_shape=jax.ShapeDtypeStruct((N_BONDS, HIDDEN), jnp.float32),
      input_output_aliases={0: 0},
  )


_round_mm_half = [_make_round_half(0), _make_round_half(1)]


def _final_body(fa_ref, an_ref, wo1_ref, wo2_ref, out_ref):
  h = jnp.dot(fa_ref[...], wo1_ref[...], preferred_element_type=jnp.float32)
  h += jnp.dot(an_ref[...], wo2_ref[...], preferred_element_type=jnp.float32)
  h = jnp.maximum(h, 0.0)
  rows = lax.broadcasted_iota(jnp.int32, (N_MOLS, N_ATOMS), 0)
  cols = lax.broadcasted_iota(jnp.int32, (N_MOLS, N_ATOMS), 1)
  pool = jnp.where(cols // ATOMS_PER_MOL == rows, 1.0 / ATOMS_PER_MOL, 0.0)
  out_ref[...] = jnp.dot(pool, h, preferred_element_type=jnp.float32)


_final_mm = pl.pallas_call(
    _final_body,
    grid=(1,),
    in_specs=[
        pl.BlockSpec((N_ATOMS, HIDDEN), lambda i: (0, 0)),
        pl.BlockSpec((N_ATOMS, HIDDEN), lambda i: (0, 0)),
        pl.BlockSpec((HIDDEN, HIDDEN), lambda i: (0, 0)),
        pl.BlockSpec((HIDDEN, HIDDEN), lambda i: (0, 0)),
    ],
    out_specs=pl.BlockSpec((N_MOLS, HIDDEN), lambda i: (0, 0)),
    out_shape=jax.ShapeDtypeStruct((N_MOLS, HIDDEN), jnp.float32),
)


def kernel(fatoms, fbonds, W_i, W_h, W_o, agraph, bgraph):
  bgt = [bgraph[:N_HALF].T, bgraph[N_HALF:].T]
  agraph_t = jnp.pad(agraph, ((0, N_ATOMS_PAD - N_ATOMS), (0, 0))).T
  gather_half = _gather_sum(N_HALF, 128, 4, "sc_gather_bonds_half")
  binput, message, dead = _binput_mm(fbonds, W_i)
  for _ in range(2):
    nei0 = gather_half(message, bgt[0])
    nei1 = gather_half(message, bgt[1])
    m0 = _round_mm_half[0](dead, binput, nei0, W_h)
    m1 = _round_mm_half[1](m0, binput, nei1, W_h)
    dead, message = message, m1
  anei = _gather_sum(N_ATOMS_PAD, 128, 4, "sc_gather_atoms")(message, agraph_t)
  return _final_mm(fatoms, anei[:N_ATOMS], W_o[:HIDDEN], W_o[HIDDEN:])


# speedup vs baseline: 1.0318x; 1.0318x over previous
import jax
import jax.numpy as jnp
from jax import lax
from jax.experimental import pallas as pl
from jax.experimental.pallas import tpu as pltpu
from jax.experimental.pallas import tpu_sc as plsc

HIDDEN = 128
MAX_NB = 16
N_ATOMS = 10000
N_BONDS = 160000
N_MOLS = 100
ATOMS_PER_MOL = N_ATOMS // N_MOLS
N_ATOMS_PAD = 10240

NC, NS = 2, 16
NW = NC * NS


def _make_gather_sum(n_rows_out, chunk, nbuf, name):
  assert n_rows_out % chunk == 0 and chunk <= 128 and chunk % 8 == 0
  total_chunks = n_rows_out // chunk
  n_iter = -(-total_chunks // NW)
  n_outer = -(-n_iter // nbuf)
  mesh = plsc.VectorSubcoreMesh(
      core_axis_name="c", subcore_axis_name="s", num_cores=NC, num_subcores=NS
  )

  def body(table_hbm, idxt_hbm, out_hbm, idx_v, acc_v, gsem, osem):
    wid = lax.axis_index("s") * NC + lax.axis_index("c")

    def outer_body(j, carry):
      for b in range(nbuf):
        c = (nbuf * j + b) * NW + wid

        @pl.when(jnp.logical_and(j > 0, c - nbuf * NW < total_chunks))
        def _(b=b):
          pltpu.make_async_copy(
              out_hbm.at[pl.ds(0, chunk)], acc_v.at[b], osem[b]
          ).wait()

        @pl.when(c < total_chunks)
        def _(b=b, c=c):
          base = c * chunk
          pltpu.sync_copy(idxt_hbm.at[:, pl.ds(base, chunk)], idx_v.at[b])
          pltpu.async_copy(
              table_hbm.at[idx_v.at[b, 0]], acc_v.at[b], gsem[b]
          ).wait()
          for k in range(1, MAX_NB):
            pltpu.async_copy(
                table_hbm.at[idx_v.at[b, k]], acc_v.at[b], gsem[b], add=True
            )

      for b in range(nbuf):
        c = (nbuf * j + b) * NW + wid

        @pl.when(c < total_chunks)
        def _(b=b, c=c):
          for _k in range(1, MAX_NB):
            pltpu.make_async_copy(
                table_hbm.at[idx_v.at[b, 0]], acc_v.at[b], gsem[b]
            ).wait()
          pltpu.async_copy(acc_v.at[b], out_hbm.at[pl.ds(c * chunk, chunk)],
                           osem[b])

      return carry

    lax.fori_loop(0, n_outer, outer_body, 0)

    for b in range(nbuf):
      c_last = (nbuf * (n_outer - 1) + b) * NW + wid

      @pl.when(c_last < total_chunks)
      def _(b=b):
        pltpu.make_async_copy(
            out_hbm.at[pl.ds(0, chunk)], acc_v.at[b], osem[b]
        ).wait()

  return pl.kernel(
      body,
      out_type=jax.ShapeDtypeStruct((n_rows_out, HIDDEN), jnp.float32),
      mesh=mesh,
      scratch_types=[
          pltpu.VMEM((nbuf, MAX_NB, chunk), jnp.int32),
          pltpu.VMEM((nbuf, chunk, HIDDEN), jnp.float32),
          [pltpu.SemaphoreType.DMA] * nbuf,
          [pltpu.SemaphoreType.DMA] * nbuf,
      ],
      name=name,
  )


_gather_cache = {}


def _gather_sum(n_rows_out, chunk, nbuf, name):
  key = (n_rows_out, chunk, nbuf, name)
  if key not in _gather_cache:
    _gather_cache[key] = _make_gather_sum(n_rows_out, chunk, nbuf, name)
  return _gather_cache[key]


_MM_ROWS = 4000


def _binput_body(fb_ref, wi_ref, bi_ref, msg_ref):
  bi = jnp.dot(fb_ref[...], wi_ref[...], preferred_element_type=jnp.float32)
  bi_ref[...] = bi
  msg_ref[...] = jnp.maximum(bi, 0.0)


_binput_mm = pl.pallas_call(
    _binput_body,
    grid=(N_BONDS // _MM_ROWS,),
    in_specs=[
        pl.BlockSpec((_MM_ROWS, HIDDEN), lambda i: (i, 0)),
        pl.BlockSpec((HIDDEN, HIDDEN), lambda i: (0, 0)),
    ],
    out_specs=[
        pl.BlockSpec((_MM_ROWS, HIDDEN), lambda i: (i, 0)),
        pl.BlockSpec((_MM_ROWS, HIDDEN), lambda i: (i, 0)),
    ],
    out_shape=[
        jax.ShapeDtypeStruct((N_BONDS, HIDDEN), jnp.float32),
        jax.ShapeDtypeStruct((N_BONDS, HIDDEN), jnp.float32),
    ],
)


def _round_body(bi_ref, nei_ref, wh_ref, out_ref):
  acc = jnp.dot(nei_ref[...], wh_ref[...], preferred_element_type=jnp.float32)
  out_ref[...] = jnp.maximum(bi_ref[...] + acc, 0.0)


_round_mm = pl.pallas_call(
    _round_body,
    grid=(N_BONDS // _MM_ROWS,),
    in_specs=[
        pl.BlockSpec((_MM_ROWS, HIDDEN), lambda i: (i, 0)),
        pl.BlockSpec((_MM_ROWS, HIDDEN), lambda i: (i, 0)),
        pl.BlockSpec((HIDDEN, HIDDEN), lambda i: (0, 0)),
    ],
    out_specs=pl.BlockSpec((_MM_ROWS, HIDDEN), lambda i: (i, 0)),
    out_shape=jax.ShapeDtypeStruct((N_BONDS, HIDDEN), jnp.float32),
)


def _final_body(fa_ref, an_ref, wo1_ref, wo2_ref, out_ref):
  h = jnp.dot(fa_ref[...], wo1_ref[...], preferred_element_type=jnp.float32)
  h += jnp.dot(an_ref[...], wo2_ref[...], preferred_element_type=jnp.float32)
  h = jnp.maximum(h, 0.0)
  rows = lax.broadcasted_iota(jnp.int32, (N_MOLS, N_ATOMS), 0)
  cols = lax.broadcasted_iota(jnp.int32, (N_MOLS, N_ATOMS), 1)
  pool = jnp.where(cols // ATOMS_PER_MOL == rows, 1.0 / ATOMS_PER_MOL, 0.0)
  out_ref[...] = jnp.dot(pool, h, preferred_element_type=jnp.float32)


_final_mm = pl.pallas_call(
    _final_body,
    grid=(1,),
    in_specs=[
        pl.BlockSpec((N_ATOMS, HIDDEN), lambda i: (0, 0)),
        pl.BlockSpec((N_ATOMS, HIDDEN), lambda i: (0, 0)),
        pl.BlockSpec((HIDDEN, HIDDEN), lambda i: (0, 0)),
        pl.BlockSpec((HIDDEN, HIDDEN), lambda i: (0, 0)),
    ],
    out_specs=pl.BlockSpec((N_MOLS, HIDDEN), lambda i: (0, 0)),
    out_shape=jax.ShapeDtypeStruct((N_MOLS, HIDDEN), jnp.float32),
)


def kernel(fatoms, fbonds, W_i, W_h, W_o, agraph, bgraph):
  bgraph_t = bgraph.T
  agraph_t = jnp.pad(agraph, ((0, N_ATOMS_PAD - N_ATOMS), (0, 0))).T
  binput, message = _binput_mm(fbonds, W_i)
  for _ in range(2):
    nei = _gather_sum(N_BONDS, 128, 6, "sc_gather_bonds")(message, bgraph_t)
    message = _round_mm(binput, nei, W_h)
  anei = _gather_sum(N_ATOMS_PAD, 128, 4, "sc_gather_atoms")(message, agraph_t)
  return _final_mm(fatoms, anei[:N_ATOMS], W_o[:HIDDEN], W_o[HIDDEN:])


# speedup vs baseline: 1.0351x; 1.0032x over previous
import jax
import jax.numpy as jnp
from jax import lax
from jax.experimental import pallas as pl
from jax.experimental.pallas import tpu as pltpu
from jax.experimental.pallas import tpu_sc as plsc

HIDDEN = 128
MAX_NB = 16
N_ATOMS = 10000
N_BONDS = 160000
N_MOLS = 100
ATOMS_PER_MOL = N_ATOMS // N_MOLS
N_ATOMS_PAD = 10240

NC, NS = 2, 16
NW = NC * NS


def _make_gather_sum(n_rows_out, chunk, nbuf, name):
  assert n_rows_out % chunk == 0 and chunk <= 128 and chunk % 8 == 0
  total_chunks = n_rows_out // chunk
  n_iter = -(-total_chunks // NW)
  n_outer = -(-n_iter // nbuf)
  mesh = plsc.VectorSubcoreMesh(
      core_axis_name="c", subcore_axis_name="s", num_cores=NC, num_subcores=NS
  )

  def body(table_hbm, idxt_hbm, out_hbm, idx_v, acc_v, gsem, osem):
    wid = lax.axis_index("s") * NC + lax.axis_index("c")

    def outer_body(j, carry):
      for b in range(nbuf):
        c = (nbuf * j + b) * NW + wid

        @pl.when(jnp.logical_and(j > 0, c - nbuf * NW < total_chunks))
        def _(b=b):
          pltpu.make_async_copy(
              out_hbm.at[pl.ds(0, chunk)], acc_v.at[b], osem[b]
          ).wait()

        @pl.when(c < total_chunks)
        def _(b=b, c=c):
          base = c * chunk
          zero16 = jnp.zeros((16,), jnp.float32)

          def zero_row(r, carry):
            for i8 in range(HIDDEN // 16):
              acc_v[b, r, pl.ds(i8 * 16, 16)] = zero16
            return carry

          lax.fori_loop(0, chunk, zero_row, 0)
          pltpu.sync_copy(idxt_hbm.at[:, pl.ds(base, chunk)], idx_v.at[b])
          for k in range(MAX_NB):
            pltpu.async_copy(
                table_hbm.at[idx_v.at[b, k]], acc_v.at[b], gsem[b], add=True
            )

      for b in range(nbuf):
        c = (nbuf * j + b) * NW + wid

        @pl.when(c < total_chunks)
        def _(b=b, c=c):
          for _k in range(MAX_NB):
            pltpu.make_async_copy(
                table_hbm.at[idx_v.at[b, 0]], acc_v.at[b], gsem[b]
            ).wait()
          pltpu.async_copy(acc_v.at[b], out_hbm.at[pl.ds(c * chunk, chunk)],
                           osem[b])

      return carry

    lax.fori_loop(0, n_outer, outer_body, 0)

    for b in range(nbuf):
      c_last = (nbuf * (n_outer - 1) + b) * NW + wid

      @pl.when(c_last < total_chunks)
      def _(b=b):
        pltpu.make_async_copy(
            out_hbm.at[pl.ds(0, chunk)], acc_v.at[b], osem[b]
        ).wait()

  return pl.kernel(
      body,
      out_type=jax.ShapeDtypeStruct((n_rows_out, HIDDEN), jnp.float32),
      mesh=mesh,
      scratch_types=[
          pltpu.VMEM((nbuf, MAX_NB, chunk), jnp.int32),
          pltpu.VMEM((nbuf, chunk, HIDDEN), jnp.float32),
          [pltpu.SemaphoreType.DMA] * nbuf,
          [pltpu.SemaphoreType.DMA] * nbuf,
      ],
      name=name,
  )


_gather_cache = {}


def _gather_sum(n_rows_out, chunk, nbuf, name):
  key = (n_rows_out, chunk, nbuf, name)
  if key not in _gather_cache:
    _gather_cache[key] = _make_gather_sum(n_rows_out, chunk, nbuf, name)
  return _gather_cache[key]


_MM_ROWS = 4000


def _binput_body(fb_ref, wi_ref, bi_ref, msg_ref):
  bi = jnp.dot(fb_ref[...], wi_ref[...], preferred_element_type=jnp.float32)
  bi_ref[...] = bi
  msg_ref[...] = jnp.maximum(bi, 0.0)


_binput_mm = pl.pallas_call(
    _binput_body,
    grid=(N_BONDS // _MM_ROWS,),
    in_specs=[
        pl.BlockSpec((_MM_ROWS, HIDDEN), lambda i: (i, 0)),
        pl.BlockSpec((HIDDEN, HIDDEN), lambda i: (0, 0)),
    ],
    out_specs=[
        pl.BlockSpec((_MM_ROWS, HIDDEN), lambda i: (i, 0)),
        pl.BlockSpec((_MM_ROWS, HIDDEN), lambda i: (i, 0)),
    ],
    out_shape=[
        jax.ShapeDtypeStruct((N_BONDS, HIDDEN), jnp.float32),
        jax.ShapeDtypeStruct((N_BONDS, HIDDEN), jnp.float32),
    ],
)


def _round_body(bi_ref, nei_ref, wh_ref, out_ref):
  acc = jnp.dot(nei_ref[...], wh_ref[...], preferred_element_type=jnp.float32)
  out_ref[...] = jnp.maximum(bi_ref[...] + acc, 0.0)


_round_mm = pl.pallas_call(
    _round_body,
    grid=(N_BONDS // _MM_ROWS,),
    in_specs=[
        pl.BlockSpec((_MM_ROWS, HIDDEN), lambda i: (i, 0)),
        pl.BlockSpec((_MM_ROWS, HIDDEN), lambda i: (i, 0)),
        pl.BlockSpec((HIDDEN, HIDDEN), lambda i: (0, 0)),
    ],
    out_specs=pl.BlockSpec((_MM_ROWS, HIDDEN), lambda i: (i, 0)),
    out_shape=jax.ShapeDtypeStruct((N_BONDS, HIDDEN), jnp.float32),
)


def _final_body(fa_ref, an_ref, wo1_ref, wo2_ref, out_ref):
  h = jnp.dot(fa_ref[...], wo1_ref[...], preferred_element_type=jnp.float32)
  h += jnp.dot(an_ref[...], wo2_ref[...], preferred_element_type=jnp.float32)
  h = jnp.maximum(h, 0.0)
  rows = lax.broadcasted_iota(jnp.int32, (N_MOLS, N_ATOMS), 0)
  cols = lax.broadcasted_iota(jnp.int32, (N_MOLS, N_ATOMS), 1)
  pool = jnp.where(cols // ATOMS_PER_MOL == rows, 1.0 / ATOMS_PER_MOL, 0.0)
  out_ref[...] = jnp.dot(pool, h, preferred_element_type=jnp.float32)


_final_mm = pl.pallas_call(
    _final_body,
    grid=(1,),
    in_specs=[
        pl.BlockSpec((N_ATOMS, HIDDEN), lambda i: (0, 0)),
        pl.BlockSpec((N_ATOMS, HIDDEN), lambda i: (0, 0)),
        pl.BlockSpec((HIDDEN, HIDDEN), lambda i: (0, 0)),
        pl.BlockSpec((HIDDEN, HIDDEN), lambda i: (0, 0)),
    ],
    out_specs=pl.BlockSpec((N_MOLS, HIDDEN), lambda i: (0, 0)),
    out_shape=jax.ShapeDtypeStruct((N_MOLS, HIDDEN), jnp.float32),
)


def kernel(fatoms, fbonds, W_i, W_h, W_o, agraph, bgraph):
  bgraph_t = bgraph.T
  agraph_t = jnp.pad(agraph, ((0, N_ATOMS_PAD - N_ATOMS), (0, 0))).T
  binput, message = _binput_mm(fbonds, W_i)
  for _ in range(2):
    nei = _gather_sum(N_BONDS, 128, 6, "sc_gather_bonds")(message, bgraph_t)
    message = _round_mm(binput, nei, W_h)
  anei = _gather_sum(N_ATOMS_PAD, 128, 4, "sc_gather_atoms")(message, agraph_t)
  return _final_mm(fatoms, anei[:N_ATOMS], W_o[:HIDDEN], W_o[HIDDEN:])


# speedup vs baseline: 1.0564x; 1.0205x over previous
import jax
import jax.numpy as jnp
from jax import lax
from jax.experimental import pallas as pl
from jax.experimental.pallas import tpu as pltpu
from jax.experimental.pallas import tpu_sc as plsc

HIDDEN = 128
MAX_NB = 16
N_ATOMS = 10000
N_BONDS = 160000
N_MOLS = 100
ATOMS_PER_MOL = N_ATOMS // N_MOLS
N_ATOMS_PAD = 10240

NC, NS = 2, 16
NW = NC * NS


def _make_gather_sum(n_rows_out, chunk, nbuf, name):
  assert n_rows_out % chunk == 0 and chunk <= 128 and chunk % 8 == 0
  total_chunks = n_rows_out // chunk
  n_iter = -(-total_chunks // NW)
  n_outer = -(-n_iter // nbuf)
  mesh = plsc.VectorSubcoreMesh(
      core_axis_name="c", subcore_axis_name="s", num_cores=NC, num_subcores=NS
  )

  def body(table_hbm, idxt_hbm, out_hbm, idx_v, acc_v, gsem, osem):
    wid = lax.axis_index("s") * NC + lax.axis_index("c")

    def outer_body(j, carry):
      for b in range(nbuf):
        c = (nbuf * j + b) * NW + wid

        @pl.when(jnp.logical_and(j > 0, c - nbuf * NW < total_chunks))
        def _(b=b):
          pltpu.make_async_copy(
              out_hbm.at[pl.ds(0, chunk)], acc_v.at[b], osem[b]
          ).wait()

        @pl.when(c < total_chunks)
        def _(b=b, c=c):
          base = c * chunk
          zero16 = jnp.zeros((16,), jnp.float32)

          def zero_row(r, carry):
            for i8 in range(HIDDEN // 16):
              acc_v[b, r, pl.ds(i8 * 16, 16)] = zero16
            return carry

          lax.fori_loop(0, chunk, zero_row, 0)
          pltpu.sync_copy(idxt_hbm.at[:, pl.ds(base, chunk)], idx_v.at[b])
          for k in range(MAX_NB):
            pltpu.async_copy(
                table_hbm.at[idx_v.at[b, k]], acc_v.at[b], gsem[b], add=True
            )

      for b in range(nbuf):
        c = (nbuf * j + b) * NW + wid

        @pl.when(c < total_chunks)
        def _(b=b, c=c):
          for _k in range(MAX_NB):
            pltpu.make_async_copy(
                table_hbm.at[idx_v.at[b, 0]], acc_v.at[b], gsem[b]
            ).wait()
          pltpu.async_copy(acc_v.at[b], out_hbm.at[pl.ds(c * chunk, chunk)],
                           osem[b])

      return carry

    lax.fori_loop(0, n_outer, outer_body, 0)

    for b in range(nbuf):
      c_last = (nbuf * (n_outer - 1) + b) * NW + wid

      @pl.when(c_last < total_chunks)
      def _(b=b):
        pltpu.make_async_copy(
            out_hbm.at[pl.ds(0, chunk)], acc_v.at[b], osem[b]
        ).wait()

  return pl.kernel(
      body,
      out_type=jax.ShapeDtypeStruct((n_rows_out, HIDDEN), jnp.float32),
      mesh=mesh,
      scratch_types=[
          pltpu.VMEM((nbuf, MAX_NB, chunk), jnp.int32),
          pltpu.VMEM((nbuf, chunk, HIDDEN), jnp.float32),
          [pltpu.SemaphoreType.DMA] * nbuf,
          [pltpu.SemaphoreType.DMA] * nbuf,
      ],
      name=name,
  )


_gather_cache = {}


def _gather_sum(n_rows_out, chunk, nbuf, name):
  key = (n_rows_out, chunk, nbuf, name)
  if key not in _gather_cache:
    _gather_cache[key] = _make_gather_sum(n_rows_out, chunk, nbuf, name)
  return _gather_cache[key]


_MM_ROWS = 4000


def _binput_body(fb_ref, wi_ref, bi_ref, msg_ref):
  bi = jnp.dot(fb_ref[...], wi_ref[...], preferred_element_type=jnp.float32)
  bi_ref[...] = bi.astype(jnp.bfloat16)
  msg_ref[...] = jnp.maximum(bi, 0.0)


_binput_mm = pl.pallas_call(
    _binput_body,
    grid=(N_BONDS // _MM_ROWS,),
    in_specs=[
        pl.BlockSpec((_MM_ROWS, HIDDEN), lambda i: (i, 0)),
        pl.BlockSpec((HIDDEN, HIDDEN), lambda i: (0, 0)),
    ],
    out_specs=[
        pl.BlockSpec((_MM_ROWS, HIDDEN), lambda i: (i, 0)),
        pl.BlockSpec((_MM_ROWS, HIDDEN), lambda i: (i, 0)),
    ],
    out_shape=[
        jax.ShapeDtypeStruct((N_BONDS, HIDDEN), jnp.bfloat16),
        jax.ShapeDtypeStruct((N_BONDS, HIDDEN), jnp.float32),
    ],
)


def _round_body(bi_ref, nei_ref, wh_ref, out_ref):
  acc = jnp.dot(nei_ref[...], wh_ref[...], preferred_element_type=jnp.float32)
  out_ref[...] = jnp.maximum(bi_ref[...].astype(jnp.float32) + acc, 0.0)


_round_mm = pl.pallas_call(
    _round_body,
    grid=(N_BONDS // _MM_ROWS,),
    in_specs=[
        pl.BlockSpec((_MM_ROWS, HIDDEN), lambda i: (i, 0)),
        pl.BlockSpec((_MM_ROWS, HIDDEN), lambda i: (i, 0)),
        pl.BlockSpec((HIDDEN, HIDDEN), lambda i: (0, 0)),
    ],
    out_specs=pl.BlockSpec((_MM_ROWS, HIDDEN), lambda i: (i, 0)),
    out_shape=jax.ShapeDtypeStruct((N_BONDS, HIDDEN), jnp.float32),
)


def _final_body(fa_ref, an_ref, wo1_ref, wo2_ref, out_ref):
  h = jnp.dot(fa_ref[...], wo1_ref[...], preferred_element_type=jnp.float32)
  h += jnp.dot(an_ref[...], wo2_ref[...], preferred_element_type=jnp.float32)
  h = jnp.maximum(h, 0.0)
  rows = lax.broadcasted_iota(jnp.int32, (N_MOLS, N_ATOMS), 0)
  cols = lax.broadcasted_iota(jnp.int32, (N_MOLS, N_ATOMS), 1)
  pool = jnp.where(cols // ATOMS_PER_MOL == rows, 1.0 / ATOMS_PER_MOL, 0.0)
  out_ref[...] = jnp.dot(pool, h, preferred_element_type=jnp.float32)


_final_mm = pl.pallas_call(
    _final_body,
    grid=(1,),
    in_specs=[
        pl.BlockSpec((N_ATOMS, HIDDEN), lambda i: (0, 0)),
        pl.BlockSpec((N_ATOMS, HIDDEN), lambda i: (0, 0)),
        pl.BlockSpec((HIDDEN, HIDDEN), lambda i: (0, 0)),
        pl.BlockSpec((HIDDEN, HIDDEN), lambda i: (0, 0)),
    ],
    out_specs=pl.BlockSpec((N_MOLS, HIDDEN), lambda i: (0, 0)),
    out_shape=jax.ShapeDtypeStruct((N_MOLS, HIDDEN), jnp.float32),
)


def kernel(fatoms, fbonds, W_i, W_h, W_o, agraph, bgraph):
  bgraph_t = bgraph.T
  agraph_t = jnp.pad(agraph, ((0, N_ATOMS_PAD - N_ATOMS), (0, 0))).T
  binput, message = _binput_mm(fbonds, W_i)
  for _ in range(2):
    nei = _gather_sum(N_BONDS, 128, 6, "sc_gather_bonds")(message, bgraph_t)
    message = _round_mm(binput, nei, W_h)
  anei = _gather_sum(N_ATOMS_PAD, 128, 4, "sc_gather_atoms")(message, agraph_t)
  return _final_mm(fatoms, anei[:N_ATOMS], W_o[:HIDDEN], W_o[HIDDEN:])


# speedup vs baseline: 1.0589x; 1.0024x over previous
import jax
import jax.numpy as jnp
from jax import lax
from jax.experimental import pallas as pl
from jax.experimental.pallas import tpu as pltpu
from jax.experimental.pallas import tpu_sc as plsc

HIDDEN = 128
MAX_NB = 16
N_ATOMS = 10000
N_BONDS = 160000
N_MOLS = 100
ATOMS_PER_MOL = N_ATOMS // N_MOLS
N_ATOMS_PAD = 10240

NC, NS = 2, 16
NW = NC * NS


def _make_gather_sum(n_rows_out, chunk, nbuf, name, split=1):
  assert n_rows_out % chunk == 0 and chunk <= 128 and chunk % 8 == 0
  sub = chunk // split
  total_chunks = n_rows_out // chunk
  n_iter = -(-total_chunks // NW)
  n_outer = -(-n_iter // nbuf)
  mesh = plsc.VectorSubcoreMesh(
      core_axis_name="c", subcore_axis_name="s", num_cores=NC, num_subcores=NS
  )

  def body(table_hbm, idxt_hbm, out_hbm, idx_v, acc_v, gsem, osem):
    wid = lax.axis_index("s") * NC + lax.axis_index("c")

    def outer_body(j, carry):
      for b in range(nbuf):
        c = (nbuf * j + b) * NW + wid

        @pl.when(jnp.logical_and(j > 0, c - nbuf * NW < total_chunks))
        def _(b=b):
          pltpu.make_async_copy(
              out_hbm.at[pl.ds(0, chunk)], acc_v.at[b], osem[b]
          ).wait()

        @pl.when(c < total_chunks)
        def _(b=b, c=c):
          base = c * chunk
          zero16 = jnp.zeros((16,), jnp.float32)

          def zero_row(r, carry):
            for i8 in range(HIDDEN // 16):
              acc_v[b, r, pl.ds(i8 * 16, 16)] = zero16
            return carry

          lax.fori_loop(0, chunk, zero_row, 0)
          pltpu.sync_copy(idxt_hbm.at[:, pl.ds(base, chunk)], idx_v.at[b])
          for k in range(MAX_NB):
            for h in range(split):
              pltpu.async_copy(
                  table_hbm.at[idx_v.at[b, k, pl.ds(h * sub, sub)]],
                  acc_v.at[b, pl.ds(h * sub, sub)], gsem[b], add=True,
              )

      for b in range(nbuf):
        c = (nbuf * j + b) * NW + wid

        @pl.when(c < total_chunks)
        def _(b=b, c=c):
          for _k in range(MAX_NB * split):
            pltpu.make_async_copy(
                table_hbm.at[idx_v.at[b, 0, pl.ds(0, sub)]],
                acc_v.at[b, pl.ds(0, sub)], gsem[b]
            ).wait()
          pltpu.async_copy(acc_v.at[b], out_hbm.at[pl.ds(c * chunk, chunk)],
                           osem[b])

      return carry

    lax.fori_loop(0, n_outer, outer_body, 0)

    for b in range(nbuf):
      c_last = (nbuf * (n_outer - 1) + b) * NW + wid

      @pl.when(c_last < total_chunks)
      def _(b=b):
        pltpu.make_async_copy(
            out_hbm.at[pl.ds(0, chunk)], acc_v.at[b], osem[b]
        ).wait()

  return pl.kernel(
      body,
      out_type=jax.ShapeDtypeStruct((n_rows_out, HIDDEN), jnp.float32),
      mesh=mesh,
      scratch_types=[
          pltpu.VMEM((nbuf, MAX_NB, chunk), jnp.int32),
          pltpu.VMEM((nbuf, chunk, HIDDEN), jnp.float32),
          [pltpu.SemaphoreType.DMA] * nbuf,
          [pltpu.SemaphoreType.DMA] * nbuf,
      ],
      name=name,
  )


_gather_cache = {}


def _gather_sum(n_rows_out, chunk, nbuf, name, split=1):
  key = (n_rows_out, chunk, nbuf, name, split)
  if key not in _gather_cache:
    _gather_cache[key] = _make_gather_sum(n_rows_out, chunk, nbuf, name, split)
  return _gather_cache[key]


_MM_ROWS = 4000


def _binput_body(fb_ref, wi_ref, bi_ref, msg_ref):
  bi = jnp.dot(fb_ref[...], wi_ref[...], preferred_element_type=jnp.float32)
  bi_ref[...] = bi.astype(jnp.bfloat16)
  msg_ref[...] = jnp.maximum(bi, 0.0)


_binput_mm = pl.pallas_call(
    _binput_body,
    grid=(N_BONDS // _MM_ROWS,),
    in_specs=[
        pl.BlockSpec((_MM_ROWS, HIDDEN), lambda i: (i, 0)),
        pl.BlockSpec((HIDDEN, HIDDEN), lambda i: (0, 0)),
    ],
    out_specs=[
        pl.BlockSpec((_MM_ROWS, HIDDEN), lambda i: (i, 0)),
        pl.BlockSpec((_MM_ROWS, HIDDEN), lambda i: (i, 0)),
    ],
    out_shape=[
        jax.ShapeDtypeStruct((N_BONDS, HIDDEN), jnp.bfloat16),
        jax.ShapeDtypeStruct((N_BONDS, HIDDEN), jnp.float32),
    ],
)


def _round_body(bi_ref, nei_ref, wh_ref, out_ref):
  acc = jnp.dot(nei_ref[...], wh_ref[...], preferred_element_type=jnp.float32)
  out_ref[...] = jnp.maximum(bi_ref[...].astype(jnp.float32) + acc, 0.0)


_round_mm = pl.pallas_call(
    _round_body,
    grid=(N_BONDS // _MM_ROWS,),
    in_specs=[
        pl.BlockSpec((_MM_ROWS, HIDDEN), lambda i: (i, 0)),
        pl.BlockSpec((_MM_ROWS, HIDDEN), lambda i: (i, 0)),
        pl.BlockSpec((HIDDEN, HIDDEN), lambda i: (0, 0)),
    ],
    out_specs=pl.BlockSpec((_MM_ROWS, HIDDEN), lambda i: (i, 0)),
    out_shape=jax.ShapeDtypeStruct((N_BONDS, HIDDEN), jnp.float32),
)


def _final_body(fa_ref, an_ref, wo1_ref, wo2_ref, out_ref):
  h = jnp.dot(fa_ref[...], wo1_ref[...], preferred_element_type=jnp.float32)
  h += jnp.dot(an_ref[...], wo2_ref[...], preferred_element_type=jnp.float32)
  h = jnp.maximum(h, 0.0)
  rows = lax.broadcasted_iota(jnp.int32, (N_MOLS, N_ATOMS), 0)
  cols = lax.broadcasted_iota(jnp.int32, (N_MOLS, N_ATOMS), 1)
  pool = jnp.where(cols // ATOMS_PER_MOL == rows, 1.0 / ATOMS_PER_MOL, 0.0)
  out_ref[...] = jnp.dot(pool, h, preferred_element_type=jnp.float32)


_final_mm = pl.pallas_call(
    _final_body,
    grid=(1,),
    in_specs=[
        pl.BlockSpec((N_ATOMS, HIDDEN), lambda i: (0, 0)),
        pl.BlockSpec((N_ATOMS, HIDDEN), lambda i: (0, 0)),
        pl.BlockSpec((HIDDEN, HIDDEN), lambda i: (0, 0)),
        pl.BlockSpec((HIDDEN, HIDDEN), lambda i: (0, 0)),
    ],
    out_specs=pl.BlockSpec((N_MOLS, HIDDEN), lambda i: (0, 0)),
    out_shape=jax.ShapeDtypeStruct((N_MOLS, HIDDEN), jnp.float32),
)


def kernel(fatoms, fbonds, W_i, W_h, W_o, agraph, bgraph):
  bgraph_t = bgraph.T
  agraph_t = jnp.pad(agraph, ((0, N_ATOMS_PAD - N_ATOMS), (0, 0))).T
  binput, message = _binput_mm(fbonds, W_i)
  for _ in range(2):
    nei = _gather_sum(N_BONDS, 128, 6, "sc_gather_bonds")(message, bgraph_t)
    message = _round_mm(binput, nei, W_h)
  anei = _gather_sum(N_ATOMS_PAD, 128, 4, "sc_gather_atoms", split=2)(message, agraph_t)
  return _final_mm(fatoms, anei[:N_ATOMS], W_o[:HIDDEN], W_o[HIDDEN:])


# speedup vs baseline: 1.0647x; 1.0055x over previous
import jax
import jax.numpy as jnp
from jax import lax
from jax.experimental import pallas as pl
from jax.experimental.pallas import tpu as pltpu
from jax.experimental.pallas import tpu_sc as plsc

HIDDEN = 128
MAX_NB = 16
N_ATOMS = 10000
N_BONDS = 160000
N_MOLS = 100
ATOMS_PER_MOL = N_ATOMS // N_MOLS
N_ATOMS_PAD = 10240

NC, NS = 2, 16
NW = NC * NS


def _make_gather_sum(n_rows_out, chunk, nbuf, name, split=1):
  assert n_rows_out % chunk == 0 and chunk <= 128 and chunk % 8 == 0
  sub = chunk // split
  total_chunks = n_rows_out // chunk
  n_iter = -(-total_chunks // NW)
  n_outer = -(-n_iter // nbuf)
  mesh = plsc.VectorSubcoreMesh(
      core_axis_name="c", subcore_axis_name="s", num_cores=NC, num_subcores=NS
  )

  def body(table_hbm, idxt_hbm, out_hbm, idx_v, acc_v, gsem, osem):
    wid = lax.axis_index("s") * NC + lax.axis_index("c")

    def outer_body(j, carry):
      for b in range(nbuf):
        c = (nbuf * j + b) * NW + wid

        @pl.when(jnp.logical_and(j > 0, c - nbuf * NW < total_chunks))
        def _(b=b):
          pltpu.make_async_copy(
              out_hbm.at[pl.ds(0, chunk)], acc_v.at[b], osem[b]
          ).wait()

        @pl.when(c < total_chunks)
        def _(b=b, c=c):
          base = c * chunk
          zero16 = jnp.zeros((16,), jnp.float32)

          def zero_row(r, carry):
            for i8 in range(HIDDEN // 16):
              acc_v[b, r, pl.ds(i8 * 16, 16)] = zero16
            return carry

          lax.fori_loop(0, chunk, zero_row, 0)
          pltpu.sync_copy(idxt_hbm.at[:, pl.ds(base, chunk)], idx_v.at[b])
          for k in range(MAX_NB):
            for h in range(split):
              pltpu.async_copy(
                  table_hbm.at[idx_v.at[b, k, pl.ds(h * sub, sub)]],
                  acc_v.at[b, pl.ds(h * sub, sub)], gsem[b], add=True,
              )

      for b in range(nbuf):
        c = (nbuf * j + b) * NW + wid

        @pl.when(c < total_chunks)
        def _(b=b, c=c):
          for _k in range(MAX_NB * split):
            pltpu.make_async_copy(
                table_hbm.at[idx_v.at[b, 0, pl.ds(0, sub)]],
                acc_v.at[b, pl.ds(0, sub)], gsem[b]
            ).wait()
          pltpu.async_copy(acc_v.at[b], out_hbm.at[pl.ds(c * chunk, chunk)],
                           osem[b])

      return carry

    lax.fori_loop(0, n_outer, outer_body, 0)

    for b in range(nbuf):
      c_last = (nbuf * (n_outer - 1) + b) * NW + wid

      @pl.when(c_last < total_chunks)
      def _(b=b):
        pltpu.make_async_copy(
            out_hbm.at[pl.ds(0, chunk)], acc_v.at[b], osem[b]
        ).wait()

  return pl.kernel(
      body,
      out_type=jax.ShapeDtypeStruct((n_rows_out, HIDDEN), jnp.float32),
      mesh=mesh,
      scratch_types=[
          pltpu.VMEM((nbuf, MAX_NB, chunk), jnp.int32),
          pltpu.VMEM((nbuf, chunk, HIDDEN), jnp.float32),
          [pltpu.SemaphoreType.DMA] * nbuf,
          [pltpu.SemaphoreType.DMA] * nbuf,
      ],
      name=name,
  )


_gather_cache = {}


def _gather_sum(n_rows_out, chunk, nbuf, name, split=1):
  key = (n_rows_out, chunk, nbuf, name, split)
  if key not in _gather_cache:
    _gather_cache[key] = _make_gather_sum(n_rows_out, chunk, nbuf, name, split)
  return _gather_cache[key]


_MM_ROWS = 4000


def _binput_body(fb_ref, wi_ref, bi_ref, msg_ref):
  bi = jnp.dot(fb_ref[...], wi_ref[...], preferred_element_type=jnp.float32)
  bi_ref[...] = bi.astype(jnp.bfloat16)
  msg_ref[...] = jnp.maximum(bi, 0.0)


_binput_mm = pl.pallas_call(
    _binput_body,
    grid=(N_BONDS // _MM_ROWS,),
    in_specs=[
        pl.BlockSpec((_MM_ROWS, HIDDEN), lambda i: (i, 0)),
        pl.BlockSpec((HIDDEN, HIDDEN), lambda i: (0, 0)),
    ],
    out_specs=[
        pl.BlockSpec((_MM_ROWS, HIDDEN), lambda i: (i, 0)),
        pl.BlockSpec((_MM_ROWS, HIDDEN), lambda i: (i, 0)),
    ],
    out_shape=[
        jax.ShapeDtypeStruct((N_BONDS, HIDDEN), jnp.bfloat16),
        jax.ShapeDtypeStruct((N_BONDS, HIDDEN), jnp.float32),
    ],
)


def _round_body(bi_ref, nei_ref, wh_ref, out_ref):
  acc = jnp.dot(nei_ref[...], wh_ref[...], preferred_element_type=jnp.float32)
  out_ref[...] = jnp.maximum(bi_ref[...].astype(jnp.float32) + acc, 0.0)


_round_mm = pl.pallas_call(
    _round_body,
    grid=(N_BONDS // _MM_ROWS,),
    in_specs=[
        pl.BlockSpec((_MM_ROWS, HIDDEN), lambda i: (i, 0)),
        pl.BlockSpec((_MM_ROWS, HIDDEN), lambda i: (i, 0)),
        pl.BlockSpec((HIDDEN, HIDDEN), lambda i: (0, 0)),
    ],
    out_specs=pl.BlockSpec((_MM_ROWS, HIDDEN), lambda i: (i, 0)),
    out_shape=jax.ShapeDtypeStruct((N_BONDS, HIDDEN), jnp.float32),
)


def _final_body(fa_ref, an_ref, wo1_ref, wo2_ref, out_ref):
  h = jnp.dot(fa_ref[...], wo1_ref[...], preferred_element_type=jnp.float32)
  h += jnp.dot(an_ref[...], wo2_ref[...], preferred_element_type=jnp.float32)
  h = jnp.maximum(h, 0.0)
  rows = lax.broadcasted_iota(jnp.int32, (N_MOLS, N_ATOMS), 0)
  cols = lax.broadcasted_iota(jnp.int32, (N_MOLS, N_ATOMS), 1)
  pool = jnp.where(cols // ATOMS_PER_MOL == rows, 1.0 / ATOMS_PER_MOL, 0.0)
  out_ref[...] = jnp.dot(pool, h, preferred_element_type=jnp.float32)


_final_mm = pl.pallas_call(
    _final_body,
    grid=(1,),
    in_specs=[
        pl.BlockSpec((N_ATOMS, HIDDEN), lambda i: (0, 0)),
        pl.BlockSpec((N_ATOMS, HIDDEN), lambda i: (0, 0)),
        pl.BlockSpec((HIDDEN, HIDDEN), lambda i: (0, 0)),
        pl.BlockSpec((HIDDEN, HIDDEN), lambda i: (0, 0)),
    ],
    out_specs=pl.BlockSpec((N_MOLS, HIDDEN), lambda i: (0, 0)),
    out_shape=jax.ShapeDtypeStruct((N_MOLS, HIDDEN), jnp.float32),
)


def kernel(fatoms, fbonds, W_i, W_h, W_o, agraph, bgraph):
  bgraph_t = bgraph.T
  agraph_t = jnp.pad(agraph, ((0, N_ATOMS_PAD - N_ATOMS), (0, 0))).T
  binput, message = _binput_mm(fbonds, W_i)
  for _ in range(2):
    nei = _gather_sum(N_BONDS, 128, 7, "sc_gather_bonds")(message, bgraph_t)
    message = _round_mm(binput, nei, W_h)
  anei = _gather_sum(N_ATOMS_PAD, 128, 4, "sc_gather_atoms", split=2)(message, agraph_t)
  return _final_mm(fatoms, anei[:N_ATOMS], W_o[:HIDDEN], W_o[HIDDEN:])


# speedup vs baseline: 1.0653x; 1.0006x over previous
import jax
import jax.numpy as jnp
from jax import lax
from jax.experimental import pallas as pl
from jax.experimental.pallas import tpu as pltpu
from jax.experimental.pallas import tpu_sc as plsc

HIDDEN = 128
MAX_NB = 16
N_ATOMS = 10000
N_BONDS = 160000
N_MOLS = 100
ATOMS_PER_MOL = N_ATOMS // N_MOLS
N_ATOMS_PAD = 10240

NC, NS = 2, 16
NW = NC * NS


def _make_gather_sum(n_rows_out, chunk, nbuf, name, split=1):
  assert n_rows_out % chunk == 0 and chunk <= 128 and chunk % 8 == 0
  sub = chunk // split
  total_chunks = n_rows_out // chunk
  n_iter = -(-total_chunks // NW)
  n_outer = -(-n_iter // nbuf)
  mesh = plsc.VectorSubcoreMesh(
      core_axis_name="c", subcore_axis_name="s", num_cores=NC, num_subcores=NS
  )

  def body(table_hbm, idxt_hbm, out_hbm, idx_v, acc_v, gsem, osem):
    wid = lax.axis_index("s") * NC + lax.axis_index("c")

    def outer_body(j, carry):
      for b in range(nbuf):
        c = (nbuf * j + b) * NW + wid

        @pl.when(jnp.logical_and(j > 0, c - nbuf * NW < total_chunks))
        def _(b=b):
          pltpu.make_async_copy(
              out_hbm.at[pl.ds(0, chunk)], acc_v.at[b], osem[b]
          ).wait()

        @pl.when(c < total_chunks)
        def _(b=b, c=c):
          base = c * chunk
          zero16 = jnp.zeros((16,), jnp.float32)

          def zero_row(r, carry):
            for i8 in range(HIDDEN // 16):
              acc_v[b, r, pl.ds(i8 * 16, 16)] = zero16
            return carry

          lax.fori_loop(0, chunk, zero_row, 0)
          pltpu.sync_copy(idxt_hbm.at[:, pl.ds(base, chunk)], idx_v.at[b])
          for k in range(MAX_NB):
            for h in range(split):
              pltpu.async_copy(
                  table_hbm.at[idx_v.at[b, k, pl.ds(h * sub, sub)]],
                  acc_v.at[b, pl.ds(h * sub, sub)], gsem[b], add=True,
              )

      for b in range(nbuf):
        c = (nbuf * j + b) * NW + wid

        @pl.when(c < total_chunks)
        def _(b=b, c=c):
          for _k in range(MAX_NB * split):
            pltpu.make_async_copy(
                table_hbm.at[idx_v.at[b, 0, pl.ds(0, sub)]],
                acc_v.at[b, pl.ds(0, sub)], gsem[b]
            ).wait()
          pltpu.async_copy(acc_v.at[b], out_hbm.at[pl.ds(c * chunk, chunk)],
                           osem[b])

      return carry

    lax.fori_loop(0, n_outer, outer_body, 0)

    for b in range(nbuf):
      c_last = (nbuf * (n_outer - 1) + b) * NW + wid

      @pl.when(c_last < total_chunks)
      def _(b=b):
        pltpu.make_async_copy(
            out_hbm.at[pl.ds(0, chunk)], acc_v.at[b], osem[b]
        ).wait()

  return pl.kernel(
      body,
      out_type=jax.ShapeDtypeStruct((n_rows_out, HIDDEN), jnp.float32),
      mesh=mesh,
      scratch_types=[
          pltpu.VMEM((nbuf, MAX_NB, chunk), jnp.int32),
          pltpu.VMEM((nbuf, chunk, HIDDEN), jnp.float32),
          [pltpu.SemaphoreType.DMA] * nbuf,
          [pltpu.SemaphoreType.DMA] * nbuf,
      ],
      name=name,
  )


_gather_cache = {}


def _gather_sum(n_rows_out, chunk, nbuf, name, split=1):
  key = (n_rows_out, chunk, nbuf, name, split)
  if key not in _gather_cache:
    _gather_cache[key] = _make_gather_sum(n_rows_out, chunk, nbuf, name, split)
  return _gather_cache[key]


_MM_ROWS = 4000


def _binput_body(fb_ref, wi_ref, bi_ref, msg_ref):
  bi = jnp.dot(fb_ref[...], wi_ref[...], preferred_element_type=jnp.float32)
  bi_ref[...] = bi.astype(jnp.bfloat16)
  msg_ref[...] = jnp.maximum(bi, 0.0)


_binput_mm = pl.pallas_call(
    _binput_body,
    grid=(N_BONDS // _MM_ROWS,),
    in_specs=[
        pl.BlockSpec((_MM_ROWS, HIDDEN), lambda i: (i, 0)),
        pl.BlockSpec((HIDDEN, HIDDEN), lambda i: (0, 0)),
    ],
    out_specs=[
        pl.BlockSpec((_MM_ROWS, HIDDEN), lambda i: (i, 0)),
        pl.BlockSpec((_MM_ROWS, HIDDEN), lambda i: (i, 0)),
    ],
    out_shape=[
        jax.ShapeDtypeStruct((N_BONDS, HIDDEN), jnp.bfloat16),
        jax.ShapeDtypeStruct((N_BONDS, HIDDEN), jnp.float32),
    ],
)


def _round_body(bi_ref, nei_ref, wh_ref, out_ref):
  acc = jnp.dot(nei_ref[...], wh_ref[...], preferred_element_type=jnp.float32)
  out_ref[...] = jnp.maximum(bi_ref[...].astype(jnp.float32) + acc, 0.0)


_round_mm = pl.pallas_call(
    _round_body,
    grid=(N_BONDS // _MM_ROWS,),
    in_specs=[
        pl.BlockSpec((_MM_ROWS, HIDDEN), lambda i: (i, 0)),
        pl.BlockSpec((_MM_ROWS, HIDDEN), lambda i: (i, 0)),
        pl.BlockSpec((HIDDEN, HIDDEN), lambda i: (0, 0)),
    ],
    out_specs=pl.BlockSpec((_MM_ROWS, HIDDEN), lambda i: (i, 0)),
    out_shape=jax.ShapeDtypeStruct((N_BONDS, HIDDEN), jnp.float32),
)


def _final_body(fa_ref, an_ref, wo1_ref, wo2_ref, out_ref):
  h = jnp.dot(fa_ref[...], wo1_ref[...], preferred_element_type=jnp.float32)
  h += jnp.dot(an_ref[...], wo2_ref[...], preferred_element_type=jnp.float32)
  h = jnp.maximum(h, 0.0)
  rows = lax.broadcasted_iota(jnp.int32, (N_MOLS, N_ATOMS), 0)
  cols = lax.broadcasted_iota(jnp.int32, (N_MOLS, N_ATOMS), 1)
  pool = jnp.where(cols // ATOMS_PER_MOL == rows, 1.0 / ATOMS_PER_MOL, 0.0)
  out_ref[...] = jnp.dot(pool, h, preferred_element_type=jnp.float32)


_final_mm = pl.pallas_call(
    _final_body,
    grid=(1,),
    in_specs=[
        pl.BlockSpec((N_ATOMS, HIDDEN), lambda i: (0, 0)),
        pl.BlockSpec((N_ATOMS, HIDDEN), lambda i: (0, 0)),
        pl.BlockSpec((HIDDEN, HIDDEN), lambda i: (0, 0)),
        pl.BlockSpec((HIDDEN, HIDDEN), lambda i: (0, 0)),
    ],
    out_specs=pl.BlockSpec((N_MOLS, HIDDEN), lambda i: (0, 0)),
    out_shape=jax.ShapeDtypeStruct((N_MOLS, HIDDEN), jnp.float32),
)


def kernel(fatoms, fbonds, W_i, W_h, W_o, agraph, bgraph):
  bgraph_t = bgraph.T
  agraph_t = jnp.pad(agraph, ((0, N_ATOMS_PAD - N_ATOMS), (0, 0))).T
  binput, message = _binput_mm(fbonds, W_i)
  for _ in range(2):
    nei = _gather_sum(N_BONDS, 128, 7, "sc_gather_bonds", split=2)(message, bgraph_t)
    message = _round_mm(binput, nei, W_h)
  anei = _gather_sum(N_ATOMS_PAD, 128, 4, "sc_gather_atoms", split=2)(message, agraph_t)
  return _final_mm(fatoms, anei[:N_ATOMS], W_o[:HIDDEN], W_o[HIDDEN:])


# speedup vs baseline: 1.0713x; 1.0056x over previous
import jax
import jax.numpy as jnp
from jax import lax
from jax.experimental import pallas as pl
from jax.experimental.pallas import tpu as pltpu
from jax.experimental.pallas import tpu_sc as plsc

HIDDEN = 128
MAX_NB = 16
N_ATOMS = 10000
N_BONDS = 160000
N_MOLS = 100
ATOMS_PER_MOL = N_ATOMS // N_MOLS
N_ATOMS_PAD = 10240

NC, NS = 2, 16
NW = NC * NS


def _make_gather_sum(n_rows_out, chunk, nbuf, name, split=1):
  assert n_rows_out % chunk == 0 and chunk <= 128 and chunk % 8 == 0
  sub = chunk // split
  total_chunks = n_rows_out // chunk
  n_iter = -(-total_chunks // NW)
  n_outer = -(-n_iter // nbuf)
  mesh = plsc.VectorSubcoreMesh(
      core_axis_name="c", subcore_axis_name="s", num_cores=NC, num_subcores=NS
  )

  def body(table_hbm, idxt_hbm, out_hbm, idx_v, acc_v, gsem, osem):
    wid = lax.axis_index("s") * NC + lax.axis_index("c")

    def outer_body(j, carry):
      for b in range(nbuf):
        c = (nbuf * j + b) * NW + wid

        @pl.when(jnp.logical_and(j > 0, c - nbuf * NW < total_chunks))
        def _(b=b):
          pltpu.make_async_copy(
              out_hbm.at[pl.ds(0, chunk)], acc_v.at[b], osem[b]
          ).wait()

        @pl.when(c < total_chunks)
        def _(b=b, c=c):
          base = c * chunk
          zero16 = jnp.zeros((16,), jnp.float32)

          def zero_row(r, carry):
            for i8 in range(HIDDEN // 16):
              acc_v[b, r, pl.ds(i8 * 16, 16)] = zero16
            return carry

          lax.fori_loop(0, chunk, zero_row, 0)
          pltpu.sync_copy(idxt_hbm.at[:, pl.ds(base, chunk)], idx_v.at[b])
          for k in range(MAX_NB):
            for h in range(split):
              pltpu.async_copy(
                  table_hbm.at[idx_v.at[b, k, pl.ds(h * sub, sub)]],
                  acc_v.at[b, pl.ds(h * sub, sub)], gsem[b], add=True,
              )

      for b in range(nbuf):
        c = (nbuf * j + b) * NW + wid

        @pl.when(c < total_chunks)
        def _(b=b, c=c):
          for _k in range(MAX_NB * split):
            pltpu.make_async_copy(
                table_hbm.at[idx_v.at[b, 0, pl.ds(0, sub)]],
                acc_v.at[b, pl.ds(0, sub)], gsem[b]
            ).wait()
          pltpu.async_copy(acc_v.at[b], out_hbm.at[pl.ds(c * chunk, chunk)],
                           osem[b])

      return carry

    lax.fori_loop(0, n_outer, outer_body, 0)

    for b in range(nbuf):
      c_last = (nbuf * (n_outer - 1) + b) * NW + wid

      @pl.when(c_last < total_chunks)
      def _(b=b):
        pltpu.make_async_copy(
            out_hbm.at[pl.ds(0, chunk)], acc_v.at[b], osem[b]
        ).wait()

  return pl.kernel(
      body,
      out_type=jax.ShapeDtypeStruct((n_rows_out, HIDDEN), jnp.float32),
      mesh=mesh,
      scratch_types=[
          pltpu.VMEM((nbuf, MAX_NB, chunk), jnp.int32),
          pltpu.VMEM((nbuf, chunk, HIDDEN), jnp.float32),
          [pltpu.SemaphoreType.DMA] * nbuf,
          [pltpu.SemaphoreType.DMA] * nbuf,
      ],
      name=name,
  )


_gather_cache = {}


def _gather_sum(n_rows_out, chunk, nbuf, name, split=1):
  key = (n_rows_out, chunk, nbuf, name, split)
  if key not in _gather_cache:
    _gather_cache[key] = _make_gather_sum(n_rows_out, chunk, nbuf, name, split)
  return _gather_cache[key]


_MM_ROWS = 8000


def _binput_body(fb_ref, wi_ref, bi_ref, msg_ref):
  bi = jnp.dot(fb_ref[...], wi_ref[...], preferred_element_type=jnp.float32)
  bi_ref[...] = bi.astype(jnp.bfloat16)
  msg_ref[...] = jnp.maximum(bi, 0.0)


_binput_mm = pl.pallas_call(
    _binput_body,
    grid=(N_BONDS // _MM_ROWS,),
    in_specs=[
        pl.BlockSpec((_MM_ROWS, HIDDEN), lambda i: (i, 0)),
        pl.BlockSpec((HIDDEN, HIDDEN), lambda i: (0, 0)),
    ],
    out_specs=[
        pl.BlockSpec((_MM_ROWS, HIDDEN), lambda i: (i, 0)),
        pl.BlockSpec((_MM_ROWS, HIDDEN), lambda i: (i, 0)),
    ],
    out_shape=[
        jax.ShapeDtypeStruct((N_BONDS, HIDDEN), jnp.bfloat16),
        jax.ShapeDtypeStruct((N_BONDS, HIDDEN), jnp.float32),
    ],
)


def _round_body(bi_ref, nei_ref, wh_ref, out_ref):
  acc = jnp.dot(nei_ref[...], wh_ref[...], preferred_element_type=jnp.float32)
  out_ref[...] = jnp.maximum(bi_ref[...].astype(jnp.float32) + acc, 0.0)


_round_mm = pl.pallas_call(
    _round_body,
    grid=(N_BONDS // _MM_ROWS,),
    in_specs=[
        pl.BlockSpec((_MM_ROWS, HIDDEN), lambda i: (i, 0)),
        pl.BlockSpec((_MM_ROWS, HIDDEN), lambda i: (i, 0)),
        pl.BlockSpec((HIDDEN, HIDDEN), lambda i: (0, 0)),
    ],
    out_specs=pl.BlockSpec((_MM_ROWS, HIDDEN), lambda i: (i, 0)),
    out_shape=jax.ShapeDtypeStruct((N_BONDS, HIDDEN), jnp.float32),
)


def _final_body(fa_ref, an_ref, wo1_ref, wo2_ref, out_ref):
  h = jnp.dot(fa_ref[...], wo1_ref[...], preferred_element_type=jnp.float32)
  h += jnp.dot(an_ref[...], wo2_ref[...], preferred_element_type=jnp.float32)
  h = jnp.maximum(h, 0.0)
  rows = lax.broadcasted_iota(jnp.int32, (N_MOLS, N_ATOMS), 0)
  cols = lax.broadcasted_iota(jnp.int32, (N_MOLS, N_ATOMS), 1)
  pool = jnp.where(cols // ATOMS_PER_MOL == rows, 1.0 / ATOMS_PER_MOL, 0.0)
  out_ref[...] = jnp.dot(pool, h, preferred_element_type=jnp.float32)


_final_mm = pl.pallas_call(
    _final_body,
    grid=(1,),
    in_specs=[
        pl.BlockSpec((N_ATOMS, HIDDEN), lambda i: (0, 0)),
        pl.BlockSpec((N_ATOMS, HIDDEN), lambda i: (0, 0)),
        pl.BlockSpec((HIDDEN, HIDDEN), lambda i: (0, 0)),
        pl.BlockSpec((HIDDEN, HIDDEN), lambda i: (0, 0)),
    ],
    out_specs=pl.BlockSpec((N_MOLS, HIDDEN), lambda i: (0, 0)),
    out_shape=jax.ShapeDtypeStruct((N_MOLS, HIDDEN), jnp.float32),
)


def kernel(fatoms, fbonds, W_i, W_h, W_o, agraph, bgraph):
  bgraph_t = bgraph.T
  agraph_t = jnp.pad(agraph, ((0, N_ATOMS_PAD - N_ATOMS), (0, 0))).T
  binput, message = _binput_mm(fbonds, W_i)
  for _ in range(2):
    nei = _gather_sum(N_BONDS, 128, 7, "sc_gather_bonds")(message, bgraph_t)
    message = _round_mm(binput, nei, W_h)
  anei = _gather_sum(N_ATOMS_PAD, 128, 4, "sc_gather_atoms", split=2)(message, agraph_t)
  return _final_mm(fatoms, anei[:N_ATOMS], W_o[:HIDDEN], W_o[HIDDEN:])


# speedup vs baseline: 1.0780x; 1.0062x over previous
import jax
import jax.numpy as jnp
from jax import lax
from jax.experimental import pallas as pl
from jax.experimental.pallas import tpu as pltpu
from jax.experimental.pallas import tpu_sc as plsc

HIDDEN = 128
MAX_NB = 16
N_ATOMS = 10000
N_BONDS = 160000
N_MOLS = 100
ATOMS_PER_MOL = N_ATOMS // N_MOLS
N_ATOMS_PAD = 10240

NC, NS = 2, 16
NW = NC * NS


def _make_gather_sum(n_rows_out, chunk, nbuf, name, split=1):
  assert n_rows_out % chunk == 0 and chunk <= 128 and chunk % 8 == 0
  sub = chunk // split
  total_chunks = n_rows_out // chunk
  n_iter = -(-total_chunks // NW)
  n_outer = -(-n_iter // nbuf)
  mesh = plsc.VectorSubcoreMesh(
      core_axis_name="c", subcore_axis_name="s", num_cores=NC, num_subcores=NS
  )

  def body(table_hbm, idxt_hbm, out_hbm, idx_v, acc_v, gsem, osem):
    wid = lax.axis_index("s") * NC + lax.axis_index("c")

    def outer_body(j, carry):
      for b in range(nbuf):
        c = (nbuf * j + b) * NW + wid

        @pl.when(jnp.logical_and(j > 0, c - nbuf * NW < total_chunks))
        def _(b=b):
          pltpu.make_async_copy(
              out_hbm.at[pl.ds(0, chunk)], acc_v.at[b], osem[b]
          ).wait()

        @pl.when(c < total_chunks)
        def _(b=b, c=c):
          base = c * chunk
          zero16 = jnp.zeros((16,), jnp.float32)

          def zero_row(r, carry):
            for i8 in range(HIDDEN // 16):
              acc_v[b, r, pl.ds(i8 * 16, 16)] = zero16
            return carry

          lax.fori_loop(0, chunk, zero_row, 0)
          pltpu.sync_copy(idxt_hbm.at[:, pl.ds(base, chunk)], idx_v.at[b])
          for k in range(MAX_NB):
            for h in range(split):
              pltpu.async_copy(
                  table_hbm.at[idx_v.at[b, k, pl.ds(h * sub, sub)]],
                  acc_v.at[b, pl.ds(h * sub, sub)], gsem[b], add=True,
              )

      for b in range(nbuf):
        c = (nbuf * j + b) * NW + wid

        @pl.when(c < total_chunks)
        def _(b=b, c=c):
          for _k in range(MAX_NB * split):
            pltpu.make_async_copy(
                table_hbm.at[idx_v.at[b, 0, pl.ds(0, sub)]],
                acc_v.at[b, pl.ds(0, sub)], gsem[b]
            ).wait()
          pltpu.async_copy(acc_v.at[b], out_hbm.at[pl.ds(c * chunk, chunk)],
                           osem[b])

      return carry

    lax.fori_loop(0, n_outer, outer_body, 0)

    for b in range(nbuf):
      c_last = (nbuf * (n_outer - 1) + b) * NW + wid

      @pl.when(c_last < total_chunks)
      def _(b=b):
        pltpu.make_async_copy(
            out_hbm.at[pl.ds(0, chunk)], acc_v.at[b], osem[b]
        ).wait()

  return pl.kernel(
      body,
      out_type=jax.ShapeDtypeStruct((n_rows_out, HIDDEN), jnp.float32),
      mesh=mesh,
      scratch_types=[
          pltpu.VMEM((nbuf, MAX_NB, chunk), jnp.int32),
          pltpu.VMEM((nbuf, chunk, HIDDEN), jnp.float32),
          [pltpu.SemaphoreType.DMA] * nbuf,
          [pltpu.SemaphoreType.DMA] * nbuf,
      ],
      name=name,
  )


_gather_cache = {}


def _gather_sum(n_rows_out, chunk, nbuf, name, split=1):
  key = (n_rows_out, chunk, nbuf, name, split)
  if key not in _gather_cache:
    _gather_cache[key] = _make_gather_sum(n_rows_out, chunk, nbuf, name, split)
  return _gather_cache[key]


_MM_ROWS = 16000


def _binput_body(fb_ref, wi_ref, bi_ref, msg_ref):
  bi = jnp.dot(fb_ref[...], wi_ref[...], preferred_element_type=jnp.float32)
  bi_ref[...] = bi.astype(jnp.bfloat16)
  msg_ref[...] = jnp.maximum(bi, 0.0)


_binput_mm = pl.pallas_call(
    _binput_body,
    grid=(N_BONDS // _MM_ROWS,),
    in_specs=[
        pl.BlockSpec((_MM_ROWS, HIDDEN), lambda i: (i, 0)),
        pl.BlockSpec((HIDDEN, HIDDEN), lambda i: (0, 0)),
    ],
    out_specs=[
        pl.BlockSpec((_MM_ROWS, HIDDEN), lambda i: (i, 0)),
        pl.BlockSpec((_MM_ROWS, HIDDEN), lambda i: (i, 0)),
    ],
    out_shape=[
        jax.ShapeDtypeStruct((N_BONDS, HIDDEN), jnp.bfloat16),
        jax.ShapeDtypeStruct((N_BONDS, HIDDEN), jnp.float32),
    ],
)


def _round_body(bi_ref, nei_ref, wh_ref, out_ref):
  acc = jnp.dot(nei_ref[...], wh_ref[...], preferred_element_type=jnp.float32)
  out_ref[...] = jnp.maximum(bi_ref[...].astype(jnp.float32) + acc, 0.0)


_round_mm = pl.pallas_call(
    _round_body,
    grid=(N_BONDS // _MM_ROWS,),
    in_specs=[
        pl.BlockSpec((_MM_ROWS, HIDDEN), lambda i: (i, 0)),
        pl.BlockSpec((_MM_ROWS, HIDDEN), lambda i: (i, 0)),
        pl.BlockSpec((HIDDEN, HIDDEN), lambda i: (0, 0)),
    ],
    out_specs=pl.BlockSpec((_MM_ROWS, HIDDEN), lambda i: (i, 0)),
    out_shape=jax.ShapeDtypeStruct((N_BONDS, HIDDEN), jnp.float32),
)


def _final_body(fa_ref, an_ref, wo1_ref, wo2_ref, out_ref):
  h = jnp.dot(fa_ref[...], wo1_ref[...], preferred_element_type=jnp.float32)
  h += jnp.dot(an_ref[...], wo2_ref[...], preferred_element_type=jnp.float32)
  h = jnp.maximum(h, 0.0)
  rows = lax.broadcasted_iota(jnp.int32, (N_MOLS, N_ATOMS), 0)
  cols = lax.broadcasted_iota(jnp.int32, (N_MOLS, N_ATOMS), 1)
  pool = jnp.where(cols // ATOMS_PER_MOL == rows, 1.0 / ATOMS_PER_MOL, 0.0)
  out_ref[...] = jnp.dot(pool, h, preferred_element_type=jnp.float32)


_final_mm = pl.pallas_call(
    _final_body,
    grid=(1,),
    in_specs=[
        pl.BlockSpec((N_ATOMS, HIDDEN), lambda i: (0, 0)),
        pl.BlockSpec((N_ATOMS, HIDDEN), lambda i: (0, 0)),
        pl.BlockSpec((HIDDEN, HIDDEN), lambda i: (0, 0)),
        pl.BlockSpec((HIDDEN, HIDDEN), lambda i: (0, 0)),
    ],
    out_specs=pl.BlockSpec((N_MOLS, HIDDEN), lambda i: (0, 0)),
    out_shape=jax.ShapeDtypeStruct((N_MOLS, HIDDEN), jnp.float32),
)


def kernel(fatoms, fbonds, W_i, W_h, W_o, agraph, bgraph):
  bgraph_t = bgraph.T
  agraph_t = jnp.pad(agraph, ((0, N_ATOMS_PAD - N_ATOMS), (0, 0))).T
  binput, message = _binput_mm(fbonds, W_i)
  for _ in range(2):
    nei = _gather_sum(N_BONDS, 128, 7, "sc_gather_bonds")(message, bgraph_t)
    message = _round_mm(binput, nei, W_h)
  anei = _gather_sum(N_ATOMS_PAD, 128, 4, "sc_gather_atoms", split=2)(message, agraph_t)
  return _final_mm(fatoms, anei[:N_ATOMS], W_o[:HIDDEN], W_o[HIDDEN:])
